# baseline (speedup 1.0000x reference)
import functools

import jax
import jax.numpy as jnp
from jax import lax
from jax.experimental import pallas as pl
from jax.experimental.pallas import tpu as pltpu
from jax.experimental.pallas import tpu_sc as plsc

N_NODES = 10000
D_IN = 128
D_EMB = 64
N_EDGES = 320000

_NC = 2
_NS = 16
_NTILE = _NC * _NS

_CH = 80
_EPAD = 327680
_EPT = _EPAD // _NTILE
_RPT = _EPT // _CH
_RPT2 = _RPT + 2
_EPT2 = _RPT2 * _CH
_NPAD = 10240
_NPS = _NPAD // _NS


def _sc_mesh():
    return plsc.VectorSubcoreMesh(core_axis_name="c", subcore_axis_name="s")


_SC_PARAMS = pltpu.CompilerParams(needs_layout_passes=False)


def _deg_body(dst_hbm, w_hbm, out_hbm, dst_v, w_v, deg_v, sem):
    del sem
    cid = lax.axis_index("c")
    sid = lax.axis_index("s")
    wid = cid * _NS + sid
    pltpu.sync_copy(dst_hbm.at[pl.ds(wid * _EPT2, _EPT2)], dst_v)
    pltpu.sync_copy(w_hbm.at[pl.ds(wid * _EPT2, _EPT2)], w_v)
    zeros = jnp.zeros((16,), jnp.float32)

    def zb(i, carry):
        deg_v[pl.ds(i * 16, 16)] = zeros
        return carry

    lax.fori_loop(0, N_NODES // 16, zb, 0)

    def chunk(j, carry):
        idx16 = dst_v[pl.ds(j * 16, 16)]
        w16 = w_v[pl.ds(j * 16, 16)]
        plsc.addupdate_scatter(deg_v, [idx16], w16)
        return carry

    lax.fori_loop(0, _EPT2 // 16, chunk, 0)
    pltpu.sync_copy(deg_v, out_hbm.at[pl.ds(wid * N_NODES, N_NODES)])


def _sc_degree(dst1, w1):
    f = pl.kernel(
        _deg_body,
        out_type=jax.ShapeDtypeStruct((_NTILE * N_NODES,), jnp.float32),
        mesh=_sc_mesh(),
        compiler_params=_SC_PARAMS,
        scratch_types=[
            pltpu.VMEM((_EPT2,), jnp.int32),
            pltpu.VMEM((_EPT2,), jnp.float32),
            pltpu.VMEM((N_NODES,), jnp.float32),
            pltpu.SemaphoreType.DMA,
        ],
    )
    return f(dst1, w1)


def _agg_body(D, ytab, src_hbm, dst3_hbm, w_hbm, out_hbm,
              src_v, dst_v, wc_v, rows_v, acc, semg0, semg1):
    cid = lax.axis_index("c")
    sid = lax.axis_index("s")
    wid = cid * _NS + sid
    base = wid * _EPT2
    pltpu.sync_copy(src_hbm.at[pl.ds(base, _EPT2)], src_v)
    pltpu.sync_copy(dst3_hbm.at[wid, pl.ds(0, _RPT)], dst_v)

    semg = (semg0, semg1)

    def _gather(j, p, sem):
        pltpu.async_copy(ytab.at[src_v.at[pl.ds(j * _CH, _CH)]],
                         rows_v.at[p], sem)
        pltpu.async_copy(w_hbm.at[pl.ds(base + j * _CH, _CH)],
                         wc_v.at[p], sem)

    def _wait_gather(j, p, sem):
        pltpu.make_async_copy(ytab.at[src_v.at[pl.ds(j * _CH, _CH)]],
                              rows_v.at[p], sem).wait()
        pltpu.make_async_copy(w_hbm.at[pl.ds(base + j * _CH, _CH)],
                              wc_v.at[p], sem).wait()

    def _scale(p):
        for g in range(_CH // 16):
            w16 = wc_v[p, pl.ds(g * 16, 16)]
            for e in range(16):
                ws = w16.at[jnp.full((16,), e, jnp.int32)].get(
                    mode="promise_in_bounds")
                idx = g * 16 + e
                for k in range(D // 16):
                    sl = pl.ds(k * 16, 16)
                    rows_v[p, idx, sl] = rows_v[p, idx, sl] * ws

    zeros = jnp.zeros((16,), jnp.float32)

    def zb(i, carry):
        for k in range(D // 16):
            rows_v[0, i, pl.ds(k * 16, 16)] = zeros
        return carry

    lax.fori_loop(0, _CH, zb, 0)
    for r in range(_NPS // _CH):
        pltpu.sync_copy(rows_v.at[0], acc.at[pl.ds(sid * _NPS + r * _CH, _CH)])
    plsc.subcore_barrier()

    _gather(0, 0, semg[0])
    _gather(1, 1, semg[1])

    def pair(t, carry):
        j0 = 2 * t
        j1 = 2 * t + 1
        _wait_gather(j0, 0, semg[0])
        _scale(0)
        pltpu.sync_copy(rows_v.at[0], acc.at[dst_v.at[j0]], add=True)
        _gather(j0 + 2, 0, semg[0])
        _wait_gather(j1, 1, semg[1])
        _scale(1)
        pltpu.sync_copy(rows_v.at[1], acc.at[dst_v.at[j1]], add=True)
        _gather(j1 + 2, 1, semg[1])
        return carry

    lax.fori_loop(0, _RPT // 2, pair, 0)

    _wait_gather(_RPT, 0, semg[0])
    _wait_gather(_RPT + 1, 1, semg[1])

    plsc.subcore_barrier()
    pltpu.sync_copy(acc.at[pl.ds(sid * _NPS, _NPS)],
                    out_hbm.at[cid, pl.ds(sid * _NPS, _NPS)])


def _sc_aggregate(ytab, src1, dst3d, w1):
    d = ytab.shape[1]
    f = pl.kernel(
        functools.partial(_agg_body, d),
        out_type=jax.ShapeDtypeStruct((_NC, _NPAD, d), jnp.float32),
        mesh=_sc_mesh(),
        compiler_params=_SC_PARAMS,
        scratch_types=[
            pltpu.VMEM((_EPT2,), jnp.int32),
            pltpu.VMEM((_RPT, _CH), jnp.int32),
            pltpu.VMEM((2, _CH), jnp.float32),
            pltpu.VMEM((2, _CH, d), jnp.float32),
            pltpu.VMEM_SHARED((_NPAD, d), jnp.float32),
            pltpu.SemaphoreType.DMA,
            pltpu.SemaphoreType.DMA,
        ],
    )
    return f(ytab, src1, dst3d, w1)


_BM = 512


def _l1_body(x_ref, w_ref, degp_ref, y_ref, dinv_ref):
    deg = jnp.sum(degp_ref[...], axis=0) + 1.0
    dinv = lax.rsqrt(deg)[:, None]
    y = jnp.dot(x_ref[...], w_ref[...], preferred_element_type=jnp.float32)
    y_ref[...] = y * dinv
    dinv_ref[...] = dinv


def _layer1(x, w1, degp):
    m, k = x.shape
    n = w1.shape[1]
    grid = (pl.cdiv(m, _BM),)
    return pl.pallas_call(
        _l1_body,
        grid=grid,
        in_specs=[
            pl.BlockSpec((_BM, k), lambda i: (i, 0)),
            pl.BlockSpec((k, n), lambda i: (0, 0)),
            pl.BlockSpec((_NTILE, _BM), lambda i: (0, i)),
        ],
        out_specs=[
            pl.BlockSpec((_BM, n), lambda i: (i, 0)),
            pl.BlockSpec((_BM, 1), lambda i: (i, 0)),
        ],
        out_shape=[
            jax.ShapeDtypeStruct((m, n), jnp.float32),
            jax.ShapeDtypeStruct((m, 1), jnp.float32),
        ],
    )(x, w1, degp)


def _l2_body(a0_ref, a1_ref, y1p_ref, dinv_ref, b1_ref, w2_ref, o_ref):
    dinv = dinv_ref[...]
    a = a0_ref[0] + a1_ref[0] + y1p_ref[...]
    h = jnp.maximum(a * dinv + b1_ref[...], 0.0)
    y = jnp.dot(h, w2_ref[...], preferred_element_type=jnp.float32) * dinv
    o_ref[...] = jnp.concatenate([y, jnp.zeros_like(y)], axis=1)


def _layer2(agg1p, y1p, dinv2d, b1, w2):
    m, k = y1p.shape
    n = w2.shape[1]
    grid = (pl.cdiv(m, _BM),)
    return pl.pallas_call(
        _l2_body,
        grid=grid,
        in_specs=[
            pl.BlockSpec((1, _BM, k), lambda i: (0, i, 0)),
            pl.BlockSpec((1, _BM, k), lambda i: (1, i, 0)),
            pl.BlockSpec((_BM, k), lambda i: (i, 0)),
            pl.BlockSpec((_BM, 1), lambda i: (i, 0)),
            pl.BlockSpec((1, k), lambda i: (0, 0)),
            pl.BlockSpec((k, n), lambda i: (0, 0)),
        ],
        out_specs=pl.BlockSpec((_BM, 2 * n), lambda i: (i, 0)),
        out_shape=jax.ShapeDtypeStruct((m, 2 * n), jnp.float32),
    )(agg1p, agg1p, y1p, dinv2d, b1, w2)


def _zfin_body(a0_ref, a1_ref, y2p_ref, dinv_ref, b2_ref, o_ref):
    a = a0_ref[0] + a1_ref[0] + y2p_ref[...]
    o_ref[...] = (a * dinv_ref[...])[:, :D_EMB] + b2_ref[...]


def _zfin(agg2p, y2p, dinv2d, b2):
    m, npad = y2p.shape
    grid = (pl.cdiv(m, _BM),)
    return pl.pallas_call(
        _zfin_body,
        grid=grid,
        in_specs=[
            pl.BlockSpec((1, _BM, npad), lambda i: (0, i, 0)),
            pl.BlockSpec((1, _BM, npad), lambda i: (1, i, 0)),
            pl.BlockSpec((_BM, npad), lambda i: (i, 0)),
            pl.BlockSpec((_BM, 1), lambda i: (i, 0)),
            pl.BlockSpec((1, D_EMB), lambda i: (0, 0)),
        ],
        out_specs=pl.BlockSpec((_BM, D_EMB), lambda i: (i, 0)),
        out_shape=jax.ShapeDtypeStruct((m, D_EMB), jnp.float32),
    )(agg2p, agg2p, y2p, dinv2d, b2)


_DEC_BM = 256
_DEC_BN = 1024


_LOG2E = 1.4426950408889634
_LN2 = 0.6931471805599453


def _dec_body(za_ref, zb_ref, bias_ref, o_ref):
    acc = lax.dot_general(
        za_ref[...], zb_ref[...], (((1,), (1,)), ((), ())),
        preferred_element_type=jnp.float32)
    x = acc + bias_ref[0]
    t = jnp.exp2(jnp.abs(x) * (-_LOG2E))
    o_ref[...] = jnp.maximum(x, 0.0) + jnp.log2(1.0 + t) * _LN2


def _decoder(z, dec_bias):
    m = z.shape[0]
    return pl.pallas_call(
        _dec_body,
        grid=(pl.cdiv(m, _DEC_BM), pl.cdiv(m, _DEC_BN)),
        in_specs=[
            pl.BlockSpec((_DEC_BM, D_EMB), lambda i, j: (i, 0)),
            pl.BlockSpec((_DEC_BN, D_EMB), lambda i, j: (j, 0)),
            pl.BlockSpec(memory_space=pltpu.SMEM),
        ],
        out_specs=pl.BlockSpec((_DEC_BM, _DEC_BN), lambda i, j: (i, j)),
        out_shape=jax.ShapeDtypeStruct((m, m), jnp.float32),
    )(z, z, dec_bias)


def kernel(x, edge_index, edge_weight, W1, b1, W2, b2, dec_bias):
    npad = _EPAD - N_EDGES

    def _slab(a, zval):
        a = jnp.concatenate([a, jnp.full((npad,), zval, a.dtype)])
        a = a.reshape(_NTILE, _RPT, _CH)
        extra = jnp.zeros((_NTILE, _RPT2 - _RPT, _CH), a.dtype)
        return jnp.concatenate([a, extra], axis=1)

    src3d = _slab(edge_index[0].astype(jnp.int32), 0)
    dst3d = _slab(edge_index[1].astype(jnp.int32), 0)
    w3d = _slab(edge_weight, 0.0)
    src1 = src3d.reshape(-1)
    dst1 = dst3d.reshape(-1)
    w1 = w3d.reshape(-1)

    degp = _sc_degree(dst1, w1).reshape(_NTILE, N_NODES)
    y1p, dinv2d = _layer1(x, W1, degp)
    agg1p = _sc_aggregate(y1p, src1, dst3d, w1)
    y2p = _layer2(agg1p, y1p, dinv2d, b1[None, :], W2)
    agg2p = _sc_aggregate(y2p, src1, dst3d, w1)
    z = _zfin(agg2p, y2p, dinv2d, b2[None, :])
    od_pred = _decoder(z, dec_bias)
    return (od_pred, z)


# speedup vs baseline: 1.2250x; 1.2250x over previous
import functools

import jax
import jax.numpy as jnp
from jax import lax
from jax.experimental import pallas as pl
from jax.experimental.pallas import tpu as pltpu
from jax.experimental.pallas import tpu_sc as plsc

N_NODES = 10000
D_IN = 128
D_EMB = 64
N_EDGES = 320000

_NC = 2
_NS = 16
_NTILE = _NC * _NS

_CH = 128
_EPAD = 327680
_EPT = _EPAD // _NTILE
_RPT = _EPT // _CH
_NPAD = 10240
_NPS = _NPAD // _NS


def _sc_mesh():
    return plsc.VectorSubcoreMesh(core_axis_name="c", subcore_axis_name="s")


_SC_PARAMS = pltpu.CompilerParams(needs_layout_passes=False)


def _deg_body(dst_hbm, w_hbm, out_hbm, dst_v, w_v, deg_v, sem):
    del sem
    cid = lax.axis_index("c")
    sid = lax.axis_index("s")
    wid = cid * _NS + sid
    pltpu.sync_copy(dst_hbm.at[pl.ds(wid * _EPT, _EPT)], dst_v)
    pltpu.sync_copy(w_hbm.at[pl.ds(wid * _EPT, _EPT)], w_v)
    zeros = jnp.zeros((16,), jnp.float32)

    def zb(i, carry):
        deg_v[pl.ds(i * 16, 16)] = zeros
        return carry

    lax.fori_loop(0, N_NODES // 16, zb, 0)

    def chunk(j, carry):
        idx16 = dst_v[pl.ds(j * 16, 16)]
        w16 = w_v[pl.ds(j * 16, 16)]
        plsc.addupdate_scatter(deg_v, [idx16], w16)
        return carry

    lax.fori_loop(0, _EPT // 16, chunk, 0)
    pltpu.sync_copy(deg_v, out_hbm.at[pl.ds(wid * N_NODES, N_NODES)])


def _sc_degree(dst1, w1):
    f = pl.kernel(
        _deg_body,
        out_type=jax.ShapeDtypeStruct((_NTILE * N_NODES,), jnp.float32),
        mesh=_sc_mesh(),
        compiler_params=_SC_PARAMS,
        scratch_types=[
            pltpu.VMEM((_EPT,), jnp.int32),
            pltpu.VMEM((_EPT,), jnp.float32),
            pltpu.VMEM((N_NODES,), jnp.float32),
            pltpu.SemaphoreType.DMA,
        ],
    )
    return f(dst1, w1)


def _agg_body(D, ytab, src_hbm, dst3_hbm, w_hbm, out_hbm,
              src_v, dst_v, w_v, rows_v, acc, sem):
    cid = lax.axis_index("c")
    sid = lax.axis_index("s")
    wid = cid * _NS + sid
    base = wid * _EPT
    pltpu.sync_copy(src_hbm.at[pl.ds(base, _EPT)], src_v)
    pltpu.sync_copy(dst3_hbm.at[wid], dst_v)
    pltpu.sync_copy(w_hbm.at[pl.ds(base, _EPT)], w_v)

    zeros = jnp.zeros((16,), jnp.float32)

    def zb(i, carry):
        for k in range(D // 16):
            rows_v[i, pl.ds(k * 16, 16)] = zeros
        return carry

    lax.fori_loop(0, _CH, zb, 0)
    for r in range(_NPS // _CH):
        pltpu.sync_copy(rows_v, acc.at[pl.ds(sid * _NPS + r * _CH, _CH)])
    plsc.subcore_barrier()

    def chunk(j, carry):
        pltpu.async_copy(ytab.at[src_v.at[pl.ds(j * _CH, _CH)]], rows_v,
                         sem).wait()
        for g in range(_CH // 16):
            w16 = w_v[pl.ds(j * _CH + g * 16, 16)]
            for e in range(16):
                ws = w16.at[jnp.full((16,), e, jnp.int32)].get(
                    mode="promise_in_bounds")
                idx = g * 16 + e
                for k in range(D // 16):
                    sl = pl.ds(k * 16, 16)
                    rows_v[idx, sl] = rows_v[idx, sl] * ws
        pltpu.sync_copy(rows_v, acc.at[dst_v.at[j]], add=True)
        return carry

    lax.fori_loop(0, _RPT, chunk, 0)
    plsc.subcore_barrier()
    pltpu.sync_copy(acc.at[pl.ds(sid * _NPS, _NPS)],
                    out_hbm.at[cid, pl.ds(sid * _NPS, _NPS)])


def _sc_aggregate(ytab, src1, dst3d, w1):
    d = ytab.shape[1]
    f = pl.kernel(
        functools.partial(_agg_body, d),
        out_type=jax.ShapeDtypeStruct((_NC, _NPAD, d), jnp.float32),
        mesh=_sc_mesh(),
        compiler_params=_SC_PARAMS,
        scratch_types=[
            pltpu.VMEM((_EPT,), jnp.int32),
            pltpu.VMEM((_RPT, _CH), jnp.int32),
            pltpu.VMEM((_EPT,), jnp.float32),
            pltpu.VMEM((_CH, d), jnp.float32),
            pltpu.VMEM_SHARED((_NPAD, d), jnp.float32),
            pltpu.SemaphoreType.DMA,
        ],
    )
    return f(ytab, src1, dst3d, w1)


_BM = 512


def _l1_body(x_ref, w_ref, degp_ref, y_ref, dinv_ref):
    deg = jnp.sum(degp_ref[...], axis=0) + 1.0
    dinv = lax.rsqrt(deg)[:, None]
    y = jnp.dot(x_ref[...], w_ref[...], preferred_element_type=jnp.float32)
    y_ref[...] = y * dinv
    dinv_ref[...] = dinv


def _layer1(x, w1, degp):
    m, k = x.shape
    n = w1.shape[1]
    grid = (pl.cdiv(m, _BM),)
    return pl.pallas_call(
        _l1_body,
        grid=grid,
        in_specs=[
            pl.BlockSpec((_BM, k), lambda i: (i, 0)),
            pl.BlockSpec((k, n), lambda i: (0, 0)),
            pl.BlockSpec((_NTILE, _BM), lambda i: (0, i)),
        ],
        out_specs=[
            pl.BlockSpec((_BM, n), lambda i: (i, 0)),
            pl.BlockSpec((_BM, 1), lambda i: (i, 0)),
        ],
        out_shape=[
            jax.ShapeDtypeStruct((m, n), jnp.float32),
            jax.ShapeDtypeStruct((m, 1), jnp.float32),
        ],
    )(x, w1, degp)


def _l2_body(a0_ref, a1_ref, y1p_ref, dinv_ref, b1_ref, w2_ref, o_ref):
    dinv = dinv_ref[...]
    a = a0_ref[0] + a1_ref[0] + y1p_ref[...]
    h = jnp.maximum(a * dinv + b1_ref[...], 0.0)
    y = jnp.dot(h, w2_ref[...], preferred_element_type=jnp.float32) * dinv
    o_ref[...] = jnp.concatenate([y, jnp.zeros_like(y)], axis=1)


def _layer2(agg1p, y1p, dinv2d, b1, w2):
    m, k = y1p.shape
    n = w2.shape[1]
    grid = (pl.cdiv(m, _BM),)
    return pl.pallas_call(
        _l2_body,
        grid=grid,
        in_specs=[
            pl.BlockSpec((1, _BM, k), lambda i: (0, i, 0)),
            pl.BlockSpec((1, _BM, k), lambda i: (1, i, 0)),
            pl.BlockSpec((_BM, k), lambda i: (i, 0)),
            pl.BlockSpec((_BM, 1), lambda i: (i, 0)),
            pl.BlockSpec((1, k), lambda i: (0, 0)),
            pl.BlockSpec((k, n), lambda i: (0, 0)),
        ],
        out_specs=pl.BlockSpec((_BM, 2 * n), lambda i: (i, 0)),
        out_shape=jax.ShapeDtypeStruct((m, 2 * n), jnp.float32),
    )(agg1p, agg1p, y1p, dinv2d, b1, w2)


def _zfin_body(a0_ref, a1_ref, y2p_ref, dinv_ref, b2_ref, o_ref):
    a = a0_ref[0] + a1_ref[0] + y2p_ref[...]
    o_ref[...] = (a * dinv_ref[...])[:, :D_EMB] + b2_ref[...]


def _zfin(agg2p, y2p, dinv2d, b2):
    m, npad = y2p.shape
    grid = (pl.cdiv(m, _BM),)
    return pl.pallas_call(
        _zfin_body,
        grid=grid,
        in_specs=[
            pl.BlockSpec((1, _BM, npad), lambda i: (0, i, 0)),
            pl.BlockSpec((1, _BM, npad), lambda i: (1, i, 0)),
            pl.BlockSpec((_BM, npad), lambda i: (i, 0)),
            pl.BlockSpec((_BM, 1), lambda i: (i, 0)),
            pl.BlockSpec((1, D_EMB), lambda i: (0, 0)),
        ],
        out_specs=pl.BlockSpec((_BM, D_EMB), lambda i: (i, 0)),
        out_shape=jax.ShapeDtypeStruct((m, D_EMB), jnp.float32),
    )(agg2p, agg2p, y2p, dinv2d, b2)


_DEC_BM = 256
_DEC_BN = 1024


_LOG2E = 1.4426950408889634
_LN2 = 0.6931471805599453


def _dec_body(za_ref, zb_ref, bias_ref, o_ref):
    acc = lax.dot_general(
        za_ref[...], zb_ref[...], (((1,), (1,)), ((), ())),
        preferred_element_type=jnp.float32)
    x = acc + bias_ref[0]
    t = jnp.exp2(jnp.abs(x) * (-_LOG2E))
    o_ref[...] = jnp.maximum(x, 0.0) + jnp.log2(1.0 + t) * _LN2


def _decoder(z, dec_bias):
    m = z.shape[0]
    return pl.pallas_call(
        _dec_body,
        grid=(pl.cdiv(m, _DEC_BM), pl.cdiv(m, _DEC_BN)),
        in_specs=[
            pl.BlockSpec((_DEC_BM, D_EMB), lambda i, j: (i, 0)),
            pl.BlockSpec((_DEC_BN, D_EMB), lambda i, j: (j, 0)),
            pl.BlockSpec(memory_space=pltpu.SMEM),
        ],
        out_specs=pl.BlockSpec((_DEC_BM, _DEC_BN), lambda i, j: (i, j)),
        out_shape=jax.ShapeDtypeStruct((m, m), jnp.float32),
    )(z, z, dec_bias)


def kernel(x, edge_index, edge_weight, W1, b1, W2, b2, dec_bias):
    npad = _EPAD - N_EDGES
    zpad_i = jnp.zeros((npad,), jnp.int32)
    src1 = jnp.concatenate([edge_index[0].astype(jnp.int32), zpad_i])
    dst1 = jnp.concatenate([edge_index[1].astype(jnp.int32), zpad_i])
    w1 = jnp.concatenate([edge_weight, jnp.zeros((npad,), jnp.float32)])
    dst3d = dst1.reshape(_NTILE, _RPT, _CH)

    degp = _sc_degree(dst1, w1).reshape(_NTILE, N_NODES)
    y1p, dinv2d = _layer1(x, W1, degp)
    agg1p = _sc_aggregate(y1p, src1, dst3d, w1)
    y2p = _layer2(agg1p, y1p, dinv2d, b1[None, :], W2)
    agg2p = _sc_aggregate(y2p, src1, dst3d, w1)
    z = _zfin(agg2p, y2p, dinv2d, b2[None, :])
    od_pred = _decoder(z, dec_bias)
    return (od_pred, z)


# speedup vs baseline: 1.9679x; 1.6064x over previous
import functools

import jax
import jax.numpy as jnp
from jax import lax
from jax.experimental import pallas as pl
from jax.experimental.pallas import tpu as pltpu
from jax.experimental.pallas import tpu_sc as plsc

N_NODES = 10000
D_IN = 128
D_EMB = 64
N_EDGES = 320000

_NC = 2
_NS = 16
_NTILE = _NC * _NS

_CH = 80
_EPT = N_EDGES // _NTILE
_RPT = _EPT // _CH
_NPAD = 10240
_NPS = _NPAD // _NS


def _sc_mesh():
    return plsc.VectorSubcoreMesh(core_axis_name="c", subcore_axis_name="s")


_SC_PARAMS = pltpu.CompilerParams(needs_layout_passes=False)


def _deg_body(dst_hbm, w_hbm, out_hbm, dst_v, w_v, deg_v, sem):
    del sem
    cid = lax.axis_index("c")
    sid = lax.axis_index("s")
    wid = cid * _NS + sid
    pltpu.sync_copy(dst_hbm.at[pl.ds(wid * _EPT, _EPT)], dst_v)
    pltpu.sync_copy(w_hbm.at[pl.ds(wid * _EPT, _EPT)], w_v)
    zeros = jnp.zeros((16,), jnp.float32)

    def zb(i, carry):
        deg_v[pl.ds(i * 16, 16)] = zeros
        return carry

    lax.fori_loop(0, N_NODES // 16, zb, 0)

    def chunk(j, carry):
        idx16 = dst_v[pl.ds(j * 16, 16)]
        w16 = w_v[pl.ds(j * 16, 16)]
        plsc.addupdate_scatter(deg_v, [idx16], w16)
        return carry

    lax.fori_loop(0, _EPT // 16, chunk, 0)
    pltpu.sync_copy(deg_v, out_hbm.at[pl.ds(wid * N_NODES, N_NODES)])


def _sc_degree(dst1, w1):
    f = pl.kernel(
        _deg_body,
        out_type=jax.ShapeDtypeStruct((_NTILE * N_NODES,), jnp.float32),
        mesh=_sc_mesh(),
        compiler_params=_SC_PARAMS,
        scratch_types=[
            pltpu.VMEM((_EPT,), jnp.int32),
            pltpu.VMEM((_EPT,), jnp.float32),
            pltpu.VMEM((N_NODES,), jnp.float32),
            pltpu.SemaphoreType.DMA,
        ],
    )
    return f(dst1, w1)


def _agg_body(D, ytab, src_hbm, dst3_hbm, w_hbm, out_hbm,
              src_v, dst_v, w_v, rows_v, acc, sem):
    cid = lax.axis_index("c")
    sid = lax.axis_index("s")
    wid = cid * _NS + sid
    base = wid * _EPT
    pltpu.sync_copy(src_hbm.at[pl.ds(base, _EPT)], src_v)
    pltpu.sync_copy(dst3_hbm.at[wid], dst_v)
    pltpu.sync_copy(w_hbm.at[pl.ds(base, _EPT)], w_v)

    zeros = jnp.zeros((16,), jnp.float32)

    def zb(i, carry):
        for k in range(D // 16):
            rows_v[i, pl.ds(k * 16, 16)] = zeros
        return carry

    lax.fori_loop(0, _CH, zb, 0)
    for r in range(_NPS // _CH):
        pltpu.sync_copy(rows_v, acc.at[pl.ds(sid * _NPS + r * _CH, _CH)])
    plsc.subcore_barrier()

    def chunk(j, carry):
        pltpu.async_copy(ytab.at[src_v.at[pl.ds(j * _CH, _CH)]], rows_v,
                         sem).wait()
        for g in range(_CH // 16):
            w16 = w_v[pl.ds(j * _CH + g * 16, 16)]
            for e in range(16):
                ws = w16.at[jnp.full((16,), e, jnp.int32)].get(
                    mode="promise_in_bounds")
                idx = g * 16 + e
                for k in range(D // 16):
                    sl = pl.ds(k * 16, 16)
                    rows_v[idx, sl] = rows_v[idx, sl] * ws
        pltpu.sync_copy(rows_v, acc.at[dst_v.at[j]], add=True)
        return carry

    lax.fori_loop(0, _RPT, chunk, 0)
    plsc.subcore_barrier()
    pltpu.sync_copy(acc.at[pl.ds(sid * _NPS, _NPS)],
                    out_hbm.at[cid, pl.ds(sid * _NPS, _NPS)])


def _sc_aggregate(ytab, src1, dst3d, w1):
    d = ytab.shape[1]
    f = pl.kernel(
        functools.partial(_agg_body, d),
        out_type=jax.ShapeDtypeStruct((_NC, _NPAD, d), jnp.float32),
        mesh=_sc_mesh(),
        compiler_params=_SC_PARAMS,
        scratch_types=[
            pltpu.VMEM((_EPT,), jnp.int32),
            pltpu.VMEM((_RPT, _CH), jnp.int32),
            pltpu.VMEM((_EPT,), jnp.float32),
            pltpu.VMEM((_CH, d), jnp.float32),
            pltpu.VMEM_SHARED((_NPAD, d), jnp.float32),
            pltpu.SemaphoreType.DMA,
        ],
    )
    return f(ytab, src1, dst3d, w1)


_BM = 512


def _l1_body(x_ref, w_ref, degp_ref, y_ref, dinv_ref):
    deg = jnp.sum(degp_ref[...], axis=0) + 1.0
    dinv = lax.rsqrt(deg)[:, None]
    y = jnp.dot(x_ref[...], w_ref[...], preferred_element_type=jnp.float32)
    y_ref[...] = y * dinv
    dinv_ref[...] = dinv


def _layer1(x, w1, degp):
    m, k = x.shape
    n = w1.shape[1]
    grid = (pl.cdiv(m, _BM),)
    return pl.pallas_call(
        _l1_body,
        grid=grid,
        in_specs=[
            pl.BlockSpec((_BM, k), lambda i: (i, 0)),
            pl.BlockSpec((k, n), lambda i: (0, 0)),
            pl.BlockSpec((_NTILE, _BM), lambda i: (0, i)),
        ],
        out_specs=[
            pl.BlockSpec((_BM, n), lambda i: (i, 0)),
            pl.BlockSpec((_BM, 1), lambda i: (i, 0)),
        ],
        out_shape=[
            jax.ShapeDtypeStruct((m, n), jnp.float32),
            jax.ShapeDtypeStruct((m, 1), jnp.float32),
        ],
    )(x, w1, degp)


def _l2_body(a0_ref, a1_ref, y1p_ref, dinv_ref, b1_ref, w2_ref, o_ref):
    dinv = dinv_ref[...]
    a = a0_ref[0] + a1_ref[0] + y1p_ref[...]
    h = jnp.maximum(a * dinv + b1_ref[...], 0.0)
    y = jnp.dot(h, w2_ref[...], preferred_element_type=jnp.float32) * dinv
    o_ref[...] = jnp.concatenate([y, jnp.zeros_like(y)], axis=1)


def _layer2(agg1p, y1p, dinv2d, b1, w2):
    m, k = y1p.shape
    n = w2.shape[1]
    grid = (pl.cdiv(m, _BM),)
    return pl.pallas_call(
        _l2_body,
        grid=grid,
        in_specs=[
            pl.BlockSpec((1, _BM, k), lambda i: (0, i, 0)),
            pl.BlockSpec((1, _BM, k), lambda i: (1, i, 0)),
            pl.BlockSpec((_BM, k), lambda i: (i, 0)),
            pl.BlockSpec((_BM, 1), lambda i: (i, 0)),
            pl.BlockSpec((1, k), lambda i: (0, 0)),
            pl.BlockSpec((k, n), lambda i: (0, 0)),
        ],
        out_specs=pl.BlockSpec((_BM, 2 * n), lambda i: (i, 0)),
        out_shape=jax.ShapeDtypeStruct((m, 2 * n), jnp.float32),
    )(agg1p, agg1p, y1p, dinv2d, b1, w2)


def _zfin_body(a0_ref, a1_ref, y2p_ref, dinv_ref, b2_ref, o_ref):
    a = a0_ref[0] + a1_ref[0] + y2p_ref[...]
    o_ref[...] = (a * dinv_ref[...])[:, :D_EMB] + b2_ref[...]


def _zfin(agg2p, y2p, dinv2d, b2):
    m, npad = y2p.shape
    grid = (pl.cdiv(m, _BM),)
    return pl.pallas_call(
        _zfin_body,
        grid=grid,
        in_specs=[
            pl.BlockSpec((1, _BM, npad), lambda i: (0, i, 0)),
            pl.BlockSpec((1, _BM, npad), lambda i: (1, i, 0)),
            pl.BlockSpec((_BM, npad), lambda i: (i, 0)),
            pl.BlockSpec((_BM, 1), lambda i: (i, 0)),
            pl.BlockSpec((1, D_EMB), lambda i: (0, 0)),
        ],
        out_specs=pl.BlockSpec((_BM, D_EMB), lambda i: (i, 0)),
        out_shape=jax.ShapeDtypeStruct((m, D_EMB), jnp.float32),
    )(agg2p, agg2p, y2p, dinv2d, b2)


_DEC_BM = 256
_DEC_BN = 1024


_LOG2E = 1.4426950408889634
_LN2 = 0.6931471805599453


def _dec_body(za_ref, zb_ref, bias_ref, o_ref):
    acc = lax.dot_general(
        za_ref[...], zb_ref[...], (((1,), (1,)), ((), ())),
        preferred_element_type=jnp.float32)
    x = acc + bias_ref[0]
    t = jnp.exp2(jnp.abs(x) * (-_LOG2E))
    o_ref[...] = jnp.maximum(x, 0.0) + jnp.log2(1.0 + t) * _LN2


def _decoder(z, dec_bias):
    m = z.shape[0]
    return pl.pallas_call(
        _dec_body,
        grid=(pl.cdiv(m, _DEC_BM), pl.cdiv(m, _DEC_BN)),
        in_specs=[
            pl.BlockSpec((_DEC_BM, D_EMB), lambda i, j: (i, 0)),
            pl.BlockSpec((_DEC_BN, D_EMB), lambda i, j: (j, 0)),
            pl.BlockSpec(memory_space=pltpu.SMEM),
        ],
        out_specs=pl.BlockSpec((_DEC_BM, _DEC_BN), lambda i, j: (i, j)),
        out_shape=jax.ShapeDtypeStruct((m, m), jnp.float32),
    )(z, z, dec_bias)


def kernel(x, edge_index, edge_weight, W1, b1, W2, b2, dec_bias):
    src1 = edge_index[0].astype(jnp.int32)
    dst1 = edge_index[1].astype(jnp.int32)
    w1 = edge_weight
    dst3d = dst1.reshape(_NTILE, _RPT, _CH)

    degp = _sc_degree(dst1, w1).reshape(_NTILE, N_NODES)
    y1p, dinv2d = _layer1(x, W1, degp)
    agg1p = _sc_aggregate(y1p, src1, dst3d, w1)
    y2p = _layer2(agg1p, y1p, dinv2d, b1[None, :], W2)
    agg2p = _sc_aggregate(y2p, src1, dst3d, w1)
    z = _zfin(agg2p, y2p, dinv2d, b2[None, :])
    od_pred = _decoder(z, dec_bias)
    return (od_pred, z)
